# baseline (speedup 1.0000x reference)
import functools

import jax
import jax.numpy as jnp
import numpy as np
from jax.experimental import pallas as pl
from jax.experimental.pallas import tpu as pltpu

VOCAB = 30522
EMBED = 512
TASK = 256
D = EMBED + TASK
NT = 8
E = 8
NHEAD = 4
DH = D // NHEAD
DHP = 256
FF = 4 * D
S = 2048
GLANES = 128

ROW_BLK = 512
N_ROW_BLKS = S // ROW_BLK
VOCAB_BLK = 512
N_VOCAB_BLKS = (VOCAB + VOCAB_BLK - 1) // VOCAB_BLK
EMB_TOK = 8
N_EMB_BLKS = S // EMB_TOK


def _sizes():
    tot = E * (E + 1) // 2
    return [max(8, int(round(FF * (i + 1) / tot))) for i in range(E)]


def _embed_kernel(ids_ref, tid_ref, *refs):
    toks = refs[:EMB_TOK]
    pos_ref, task_ref, out_ref = refs[EMB_TOK], refs[EMB_TOK + 1], refs[EMB_TOK + 2]
    tok = jnp.concatenate([t[0] for t in toks], axis=0)
    out_ref[0, :, :EMBED] = tok + pos_ref[0]
    out_ref[0, :, EMBED:] = jnp.broadcast_to(task_ref[0], (EMB_TOK, TASK))


def _embed(ids, task_id, tok3, pos3, task_embed):
    tok_specs = [
        pl.BlockSpec((1, 1, EMBED),
                     functools.partial(lambda j, i, ids, tid: (ids[EMB_TOK * i + j], 0, 0), j))
        for j in range(EMB_TOK)
    ]
    grid_spec = pltpu.PrefetchScalarGridSpec(
        num_scalar_prefetch=2,
        grid=(N_EMB_BLKS,),
        in_specs=tok_specs + [
            pl.BlockSpec((1, EMB_TOK, EMBED), lambda i, ids, tid: (i, 0, 0)),
            pl.BlockSpec((1, 1, TASK), lambda i, ids, tid: (tid[0], 0, 0)),
        ],
        out_specs=pl.BlockSpec((1, EMB_TOK, D), lambda i, ids, tid: (i, 0, 0)),
    )
    out = pl.pallas_call(
        _embed_kernel,
        grid_spec=grid_spec,
        out_shape=jax.ShapeDtypeStruct((N_EMB_BLKS, EMB_TOK, D), jnp.float32),
    )(ids, task_id, *([tok3] * EMB_TOK), pos3, task_embed.reshape(NT, 1, TASK))
    return out.reshape(S, D)


def _qkv_kernel(x_ref, w_ref, b_ref, o_ref):
    o_ref[...] = jnp.dot(x_ref[...], w_ref[...],
                         preferred_element_type=jnp.float32) + b_ref[...]


def _qkv(x, wqkv, bqkv):
    w3 = 3 * NHEAD * DHP
    return pl.pallas_call(
        _qkv_kernel,
        grid=(N_ROW_BLKS,),
        in_specs=[
            pl.BlockSpec((ROW_BLK, D), lambda i: (i, 0)),
            pl.BlockSpec((D, w3), lambda i: (0, 0)),
            pl.BlockSpec((1, w3), lambda i: (0, 0)),
        ],
        out_specs=pl.BlockSpec((ROW_BLK, w3), lambda i: (i, 0)),
        out_shape=jax.ShapeDtypeStruct((S, w3), jnp.float32),
    )(x, wqkv, bqkv)


KV_BLK = 1024


def _attn_kernel(q_ref, k_ref, v_ref, o_ref):
    q = q_ref[...]
    scale = np.float32(1.0 / np.sqrt(float(DH)))
    nrows = q.shape[0]
    m = jnp.full((nrows, 1), -jnp.inf, jnp.float32)
    l = jnp.zeros((nrows, 1), jnp.float32)
    o = jnp.zeros((nrows, DHP), jnp.float32)
    for j in range(S // KV_BLK):
        k_blk = k_ref[pl.ds(j * KV_BLK, KV_BLK), :]
        v_blk = v_ref[pl.ds(j * KV_BLK, KV_BLK), :]
        s = jax.lax.dot_general(q, k_blk, (((1,), (1,)), ((), ())),
                                preferred_element_type=jnp.float32) * scale
        m_blk = jnp.max(s, axis=-1, keepdims=True)
        m_new = jnp.maximum(m, m_blk)
        corr = jnp.where(m == m_new, 0.0, m - m_new)
        e = jnp.exp(s - m_new)
        l_blk = jnp.sum(e, axis=-1, keepdims=True)
        ec_l = jnp.exp(corr) * l
        l_new = ec_l + l_blk
        acc0 = ec_l * o
        o = jnp.dot(e, v_blk, preferred_element_type=jnp.float32) + acc0
        o = o * (1.0 / l_new)
        m, l = m_new, l_new
    o_ref[...] = o


def _attention(qkv):
    return pl.pallas_call(
        _attn_kernel,
        grid=(NHEAD, N_ROW_BLKS),
        in_specs=[
            pl.BlockSpec((ROW_BLK, DHP), lambda h, i: (i, h)),
            pl.BlockSpec((S, DHP), lambda h, i: (0, NHEAD + h)),
            pl.BlockSpec((S, DHP), lambda h, i: (0, 2 * NHEAD + h)),
        ],
        out_specs=pl.BlockSpec((ROW_BLK, DHP), lambda h, i: (i, h)),
        out_shape=jax.ShapeDtypeStruct((S, NHEAD * DHP), jnp.float32),
    )(qkv, qkv, qkv)


def _ln(x, g, b):
    m = jnp.mean(x, axis=-1, keepdims=True)
    v = jnp.mean((x - m) ** 2, axis=-1, keepdims=True)
    return (x - m) / jnp.sqrt(v + 1e-5) * g + b


def _postattn_kernel(attn_ref, resid_ref, wo_ref, bo_ref, g1_ref, b1_ref,
                     xn_ref):
    o = jnp.dot(attn_ref[...], wo_ref[...],
                preferred_element_type=jnp.float32) + bo_ref[...]
    x1 = resid_ref[...] + o
    xn_ref[...] = _ln(x1, g1_ref[...], b1_ref[...])


def _gate_kernel(xn_ref, gw_ref, gb_ref, probs_ref, gates_ref, em_ref):
    logits = jnp.dot(xn_ref[...], gw_ref[...],
                     preferred_element_type=jnp.float32) + gb_ref[...]
    lane = jax.lax.broadcasted_iota(jnp.int32, logits.shape, 1)
    valid = lane < E
    logits = jnp.where(valid, logits, -jnp.inf)
    m = jnp.max(logits, axis=-1, keepdims=True)
    ex = jnp.where(valid, jnp.exp(logits - m), 0.0)
    probs = ex / jnp.sum(ex, axis=-1, keepdims=True)
    probs_ref[...] = probs

    m1 = jnp.max(probs, axis=-1, keepdims=True)
    i1 = jnp.min(jnp.where((probs == m1) & valid, lane, GLANES), axis=-1, keepdims=True)
    rest = jnp.where(lane == i1, -1.0, probs)
    m2 = jnp.max(rest, axis=-1, keepdims=True)
    i2 = jnp.min(jnp.where((rest == m2) & valid, lane, GLANES), axis=-1, keepdims=True)
    den = m1 + m2
    gates_ref[...] = (jnp.where(lane == i1, m1 / den, 0.0)
                      + jnp.where(lane == i2, m2 / den, 0.0))
    em_ref[...] = jnp.where((lane == i1) | (lane == i2), 1.0, 0.0)


def _postattn(attn, resid, wo, bo, ln_g, ln_b):
    return pl.pallas_call(
        _postattn_kernel,
        grid=(N_ROW_BLKS,),
        in_specs=[
            pl.BlockSpec((ROW_BLK, NHEAD * DHP), lambda i: (i, 0)),
            pl.BlockSpec((ROW_BLK, D), lambda i: (i, 0)),
            pl.BlockSpec((NHEAD * DHP, D), lambda i: (0, 0)),
            pl.BlockSpec((1, D), lambda i: (0, 0)),
            pl.BlockSpec((1, D), lambda i: (0, 0)),
            pl.BlockSpec((1, D), lambda i: (0, 0)),
        ],
        out_specs=pl.BlockSpec((ROW_BLK, D), lambda i: (i, 0)),
        out_shape=jax.ShapeDtypeStruct((S, D), jnp.float32),
    )(attn, resid, wo, bo, ln_g, ln_b)


def _gate(xn, gw_pad, gb_eff):
    shp = jax.ShapeDtypeStruct
    return pl.pallas_call(
        _gate_kernel,
        grid=(N_ROW_BLKS,),
        in_specs=[
            pl.BlockSpec((ROW_BLK, D), lambda i: (i, 0)),
            pl.BlockSpec((D, GLANES), lambda i: (0, 0)),
            pl.BlockSpec((1, GLANES), lambda i: (0, 0)),
        ],
        out_specs=[
            pl.BlockSpec((ROW_BLK, GLANES), lambda i: (i, 0)),
            pl.BlockSpec((ROW_BLK, GLANES), lambda i: (i, 0)),
            pl.BlockSpec((ROW_BLK, GLANES), lambda i: (i, 0)),
        ],
        out_shape=[
            shp((S, GLANES), jnp.float32),
            shp((S, GLANES), jnp.float32),
            shp((S, GLANES), jnp.float32),
        ],
    )(xn, gw_pad, gb_eff)


def _moe_h_kernel(xn_ref, *refs):
    xn = xn_ref[...]
    for i in range(E):
        w1, b1, h_ref = refs[2 * i], refs[2 * i + 1], refs[2 * E + i]
        h_ref[...] = jax.nn.relu(
            jnp.dot(xn, w1[...], preferred_element_type=jnp.float32) + b1[...])


def _moe_out_kernel(xn_ref, g_ref, g2_ref, b2_ref, *refs):
    hs = refs[:E]
    ew = refs[E:3 * E]
    out_ref = refs[3 * E]
    xn = xn_ref[...]
    acc = jnp.zeros(xn.shape, jnp.float32)
    for i in range(E):
        w2, b2 = ew[2 * i], ew[2 * i + 1]
        y = jnp.dot(hs[i][...], w2[...], preferred_element_type=jnp.float32) + b2[...]
        acc = acc + g_ref[:, i:i + 1] * y
    out_ref[...] = _ln(xn + acc, g2_ref[...], b2_ref[...])


def _moe(xn, gates, ln_g, ln_b, expert_ws):
    sizes = _sizes()
    shp = jax.ShapeDtypeStruct
    w1specs, w2specs, hspecs = [], [], []
    for sz in sizes:
        w1specs += [pl.BlockSpec((D, sz), lambda i: (0, 0)),
                    pl.BlockSpec((1, sz), lambda i: (0, 0))]
        w2specs += [pl.BlockSpec((sz, D), lambda i: (0, 0)),
                    pl.BlockSpec((1, D), lambda i: (0, 0))]
        hspecs.append(pl.BlockSpec((ROW_BLK, sz), lambda i: (i, 0)))
    w1s = [expert_ws[4 * i + j] for i in range(E) for j in (0, 1)]
    w2s = [expert_ws[4 * i + j] for i in range(E) for j in (2, 3)]
    hs = pl.pallas_call(
        _moe_h_kernel,
        grid=(N_ROW_BLKS,),
        in_specs=[pl.BlockSpec((ROW_BLK, D), lambda i: (i, 0))] + w1specs,
        out_specs=hspecs,
        out_shape=[shp((S, sz), jnp.float32) for sz in sizes],
    )(xn, *w1s)
    return pl.pallas_call(
        _moe_out_kernel,
        grid=(N_ROW_BLKS,),
        in_specs=[
            pl.BlockSpec((ROW_BLK, D), lambda i: (i, 0)),
            pl.BlockSpec((ROW_BLK, GLANES), lambda i: (i, 0)),
            pl.BlockSpec((1, D), lambda i: (0, 0)),
            pl.BlockSpec((1, D), lambda i: (0, 0)),
        ] + hspecs + w2specs,
        out_specs=pl.BlockSpec((ROW_BLK, D), lambda i: (i, 0)),
        out_shape=shp((S, D), jnp.float32),
    )(xn, gates, ln_g, ln_b, *hs, *w2s)


def _channel_kernel(x_ref, noise_ref, snr_ref, mask_ref, rx_ref, len_ref):
    x = x_ref[...]
    snr_lin = jnp.exp((snr_ref[0, 0] / 10.0) * np.float32(np.log(10.0)))
    sig_p = jnp.mean(x * x)
    sigma = jnp.sqrt(sig_p / snr_lin)
    rx_ref[...] = x + sigma * noise_ref[...]
    len_ref[...] = jnp.sum(mask_ref[...], axis=(0, 1), keepdims=True)


def _channel(x, noise, snr, mask):
    shp = jax.ShapeDtypeStruct
    return pl.pallas_call(
        _channel_kernel,
        out_shape=[shp((S, D), jnp.float32), shp((1, 1), jnp.float32)],
    )(x, noise, snr, mask)


def _head_kernel(x_ref, w_ref, b_ref, o_ref):
    o_ref[...] = jnp.dot(x_ref[...], w_ref[...],
                         preferred_element_type=jnp.float32) + b_ref[...]


def _head(x, head_w, head_b):
    return pl.pallas_call(
        _head_kernel,
        grid=(N_VOCAB_BLKS,),
        in_specs=[
            pl.BlockSpec((S, D), lambda j: (0, 0)),
            pl.BlockSpec((D, VOCAB_BLK), lambda j: (0, j)),
            pl.BlockSpec((1, VOCAB_BLK), lambda j: (0, j)),
        ],
        out_specs=pl.BlockSpec((S, VOCAB_BLK), lambda j: (0, j)),
        out_shape=jax.ShapeDtypeStruct((S, VOCAB), jnp.float32),
    )(x, head_w, head_b)


def _layer(x, p, snr):
    wqkv = jnp.concatenate([p["wq"], p["wk"], p["wv"]], axis=1)
    wqkv = (jnp.zeros((D, 3, NHEAD, DHP), jnp.float32)
            .at[:, :, :, :DH].set(wqkv.reshape(D, 3, NHEAD, DH))
            .reshape(D, 3 * NHEAD * DHP))
    bqkv = jnp.concatenate([p["bq"], p["bk"], p["bv"]])
    bqkv = (jnp.zeros((3, NHEAD, DHP), jnp.float32)
            .at[:, :, :DH].set(bqkv.reshape(3, NHEAD, DH))
            .reshape(1, 3 * NHEAD * DHP))
    wo = (jnp.zeros((NHEAD, DHP, D), jnp.float32)
          .at[:, :DH, :].set(p["wo"].reshape(NHEAD, DH, D))
          .reshape(NHEAD * DHP, D))
    gw_pad = jnp.zeros((D, GLANES), jnp.float32).at[:, :E].set(p["gate_w"][:D])
    snr_b = snr[0].astype(jnp.bfloat16).astype(jnp.float32)
    gwl_b = p["gate_w"][D].astype(jnp.bfloat16).astype(jnp.float32)
    gb_eff = jnp.zeros((1, GLANES), jnp.float32).at[0, :E].set(
        p["gate_b"] + snr_b * gwl_b)

    qkv = _qkv(x, wqkv, bqkv)
    attn = _attention(qkv)
    xn = _postattn(attn, x, wo, p["bo"][None, :], p["ln1_g"][None, :],
                   p["ln1_b"][None, :])
    probs, gates, emask = _gate(xn, gw_pad, gb_eff)
    expert_ws = []
    for ep in p["experts"]:
        expert_ws += [ep["w1"], ep["b1"][None, :], ep["w2"], ep["b2"][None, :]]
    x2 = _moe(xn, gates, p["ln2_g"][None, :], p["ln2_b"][None, :], expert_ws)
    return x2, probs[:, :E], emask[:, :E]


def kernel(input_ids, attn_mask, task_id, snr, params):
    ids = input_ids.reshape(-1).astype(jnp.int32)
    tid = task_id.reshape(-1).astype(jnp.int32)
    tok3 = params["tok_embed"].reshape(VOCAB, 1, EMBED)
    pos3 = params["pos_embed"].reshape(N_EMB_BLKS, EMB_TOK, EMBED)

    x = _embed(ids, tid, tok3, pos3, params["task_embed"])

    gss, ems = [], []
    for p in params["enc"]:
        x, gs, em = _layer(x, p, snr)
        gss.append(gs)
        ems.append(em)
    enc = x

    noise = jax.random.normal(jax.random.key(1234), (1, S, D), jnp.float32)
    rx, length = _channel(enc, noise.reshape(S, D), snr.reshape(1, 1),
                          attn_mask.astype(jnp.float32))

    x = rx
    for p in params["dec"]:
        x, gs, em = _layer(x, p, snr)
        gss.append(gs)
        ems.append(em)
    dec = x

    logits = _head(dec, params["head_w"], params["head_b"][None, :])

    return (logits.reshape(1, S, VOCAB), input_ids, length.reshape(1),
            enc.reshape(1, S, D), dec.reshape(1, S, D),
            jnp.stack(gss), jnp.stack(ems))
